# baseline (speedup 1.0000x reference)
import functools

import jax
import jax.numpy as jnp
from jax import lax
from jax.experimental import pallas as pl
from jax.experimental.pallas import tpu as pltpu
from jax.experimental.pallas import tpu_sc as plsc

_SCALE = 2048 ** -0.5
_BLK = 1024

_S = 8192
_D = 2048
_NC = 2
_NS = 16
_NW = _NC * _NS
_SC_ROWS = 3072
_TC_ROWS = _S - _SC_ROWS
_ROWS_W = _SC_ROWS // _NW
_CH = 8
_NCH = _ROWS_W // _CH
_NBUF = 6
_AHEAD = _NBUF - 2
_UNROLL = 8


def _tc_scale_copy(emb_ref, o_ref):
    o_ref[...] = emb_ref[...] * _SCALE


def _tc_kernel(emb):
    d = emb.shape[1]
    return pl.pallas_call(
        _tc_scale_copy,
        grid=(_TC_ROWS // _BLK,),
        in_specs=[pl.BlockSpec((_BLK, d), lambda i: (i, 0))],
        out_specs=pl.BlockSpec((_BLK, d), lambda i: (i, 0)),
        out_shape=jax.ShapeDtypeStruct((_TC_ROWS, d), emb.dtype),
    )(emb)


def _sc_scale_body(emb_hbm, out_hbm, *rest):
    bufs = rest[:_NBUF]
    gsems = rest[_NBUF:2 * _NBUF]
    ssems = rest[2 * _NBUF:3 * _NBUF]
    wid = lax.axis_index("s") * _NC + lax.axis_index("c")
    row0 = wid * _ROWS_W

    def src(g):
        return emb_hbm.at[pl.ds(_TC_ROWS + row0 + g * _CH, _CH), :]

    def dst(g):
        return out_hbm.at[pl.ds(row0 + g * _CH, _CH), :]

    gathers = {}
    scatters = {}
    waited = set()
    for g in range(min(_AHEAD, _NCH)):
        gathers[g] = pltpu.async_copy(src(g), bufs[g % _NBUF], gsems[g % _NBUF])
    for g in range(_NCH):
        b = g % _NBUF
        gathers[g].wait()
        buf = bufs[b]

        @plsc.parallel_loop(0, _CH * _D, step=16, unroll=_UNROLL)
        def _mul_body(i, buf=buf):
            r = lax.shift_right_logical(i, 11)
            c = lax.bitwise_and(i, _D - 1)
            sl = pl.ds(pl.multiple_of(c, 16), 16)
            buf[r, sl] = buf[r, sl] * _SCALE

        scatters[g] = pltpu.async_copy(buf, dst(g), ssems[b])
        nxt = g + _AHEAD
        if nxt < _NCH:
            prev = nxt - _NBUF
            if prev >= 0:
                scatters[prev].wait()
                waited.add(prev)
            gathers[nxt] = pltpu.async_copy(
                src(nxt), bufs[nxt % _NBUF], gsems[nxt % _NBUF]
            )
    for g in range(_NCH):
        if g not in waited:
            scatters[g].wait()


@functools.lru_cache(maxsize=None)
def _sc_scale_kernel():
    return pl.kernel(
        _sc_scale_body,
        mesh=plsc.VectorSubcoreMesh(
            core_axis_name="c", subcore_axis_name="s"
        ),
        out_type=jax.ShapeDtypeStruct((_SC_ROWS, _D), jnp.float32),
        scratch_types=(
            [pltpu.VMEM((_CH, _D), jnp.float32)] * _NBUF
            + [pltpu.SemaphoreType.DMA] * (2 * _NBUF)
        ),
    )


def kernel(x, emb):
    sc_out = _sc_scale_kernel()(emb)
    tc_out = _tc_kernel(emb)
    return jnp.concatenate([tc_out, sc_out], axis=0)[None]


# speedup vs baseline: 1.4872x; 1.4872x over previous
import functools

import jax
import jax.numpy as jnp
from jax import lax
from jax.experimental import pallas as pl
from jax.experimental.pallas import tpu as pltpu
from jax.experimental.pallas import tpu_sc as plsc

_SCALE = 2048 ** -0.5

_S = 8192
_D = 2048
_NC = 2
_NS = 16
_NW = _NC * _NS
_ROWS_W = _S // _NW
_CH = 8
_NCH = _ROWS_W // _CH
_NBUF = 6
_AHEAD = _NBUF - 2
_UNROLL = 8


def _sc_scale_body(emb_hbm, out_hbm, *rest):
    bufs = rest[:_NBUF]
    gsems = rest[_NBUF:2 * _NBUF]
    ssems = rest[2 * _NBUF:3 * _NBUF]
    wid = lax.axis_index("s") * _NC + lax.axis_index("c")
    row0 = wid * _ROWS_W

    def src(g):
        return emb_hbm.at[pl.ds(row0 + g * _CH, _CH), :]

    def dst(g):
        return out_hbm.at[pl.ds(row0 + g * _CH, _CH), :]

    gathers = {}
    scatters = {}
    waited = set()
    for g in range(min(_AHEAD, _NCH)):
        gathers[g] = pltpu.async_copy(src(g), bufs[g % _NBUF], gsems[g % _NBUF])
    for g in range(_NCH):
        b = g % _NBUF
        gathers[g].wait()
        buf = bufs[b]

        @plsc.parallel_loop(0, _CH * _D, step=16, unroll=_UNROLL)
        def _mul_body(i, buf=buf):
            r = lax.shift_right_logical(i, 11)
            c = lax.bitwise_and(i, _D - 1)
            sl = pl.ds(pl.multiple_of(c, 16), 16)
            buf[r, sl] = buf[r, sl] * _SCALE

        scatters[g] = pltpu.async_copy(buf, dst(g), ssems[b])
        nxt = g + _AHEAD
        if nxt < _NCH:
            prev = nxt - _NBUF
            if prev >= 0:
                scatters[prev].wait()
                waited.add(prev)
            gathers[nxt] = pltpu.async_copy(
                src(nxt), bufs[nxt % _NBUF], gsems[nxt % _NBUF]
            )
    for g in range(_NCH):
        if g not in waited:
            scatters[g].wait()


@functools.lru_cache(maxsize=None)
def _sc_scale_kernel():
    return pl.kernel(
        _sc_scale_body,
        mesh=plsc.VectorSubcoreMesh(
            core_axis_name="c", subcore_axis_name="s"
        ),
        out_type=jax.ShapeDtypeStruct((_S, _D), jnp.float32),
        scratch_types=(
            [pltpu.VMEM((_CH, _D), jnp.float32)] * _NBUF
            + [pltpu.SemaphoreType.DMA] * (2 * _NBUF)
        ),
    )


def kernel(x, emb):
    out = _sc_scale_kernel()(emb)
    return out[None]
